# Initial kernel scaffold; baseline (speedup 1.0000x reference)
import jax
import jax.numpy as jnp
from jax.experimental import pallas as pl


def kernel(all_tubes, gt_boxes, num_boxes):
    raise NotImplementedError("write your pallas kernel here")


# speedup vs baseline: 2.0217x; 2.0217x over previous
import jax
import jax.numpy as jnp
from jax import lax
from jax.experimental import pallas as pl
from jax.experimental.pallas import tpu as pltpu
from jax.experimental.pallas import tpu_sc as plsc

L = 16
B = 8
N_PROP = 5000
G = 40
N_TOT = N_PROP + G
N_PAD = 5056
CHUNK = N_PAD // 4
NV_CHUNK = CHUNK // L
NV_FULL = N_PAD // L
NL1 = 320
K_FG = 32
K_BG = 96
K_ALL = 128
FG_TH = 0.5
BG_HI = 0.5
BG_LO = 0.1
STDS = (0.1, 0.1, 0.1, 0.2, 0.2, 0.2)
BIG = 10 ** 6
GT_FLAT = G * 8 + 16


def _ln(x):
    bits = lax.bitcast_convert_type(x, jnp.int32)
    e = ((bits >> 23) & 0xFF) - 127
    m = lax.bitcast_convert_type((bits & 0x007FFFFF) | 0x3F800000, jnp.float32)
    t = (m - 1.0) / (m + 1.0)
    t2 = t * t
    p = t * (2.0 + t2 * (2.0 / 3.0 + t2 * (2.0 / 5.0 + t2 * (2.0 / 7.0
                                                             + t2 * (2.0 / 9.0)))))
    return e.astype(jnp.float32) * 0.6931471805599453 + p


def _body(coords_hbm, gt_hbm, rois_out, lab_out, tgt_out, w_out,
          s_cvm, s_max_c, s_asn_c, sp_max, sp_asn,
          s_coord, s_gt, s_max, s_asn, s_fg, s_bg,
          s_l1fg, s_l1bg, s_selidx, s_slotfg,
          s_rois, s_lab, s_tgt, s_w, s_tmp):
    c = lax.axis_index("c")
    s = lax.axis_index("s")
    b = c * 4 + s // 4
    bl = s // 4
    ch = s % 4
    base = ch * CHUNK
    iota = lax.iota(jnp.int32, L)
    lane0 = iota == 0

    def put1(ref, pos, scalar_val):
        plsc.store_scatter(ref, [jnp.full((L,), 0, jnp.int32) + pos],
                           jnp.zeros((L,), ref.dtype) + scalar_val, mask=lane0)

    pltpu.sync_copy(gt_hbm.at[pl.ds(b * (G * 8), G * 8)],
                    s_gt.at[pl.ds(0, G * 8)])
    for j in range(6):
        pltpu.sync_copy(
            coords_hbm.at[pl.ds(b * (6 * N_PAD) + j * N_PAD + base, CHUNK)],
            s_cvm.at[pl.ds(j * CHUNK, CHUNK)])

    def v_body(v, _):
        x1 = s_cvm[pl.ds(0 * CHUNK + v * L, L)]
        y1 = s_cvm[pl.ds(1 * CHUNK + v * L, L)]
        t1 = s_cvm[pl.ds(2 * CHUNK + v * L, L)]
        x2 = s_cvm[pl.ds(3 * CHUNK + v * L, L)]
        y2 = s_cvm[pl.ds(4 * CHUNK + v * L, L)]
        t2 = s_cvm[pl.ds(5 * CHUNK + v * L, L)]
        vol_r = (x2 - x1 + 1.0) * (y2 - y1 + 1.0) * (t2 - t1 + 1.0)

        def g_body(g, carry):
            mv, av = carry
            gvec = s_gt[pl.ds(g * 8, L)]
            g0 = gvec[0]
            g1 = gvec[1]
            g2 = gvec[2]
            g3 = gvec[3]
            g4 = gvec[4]
            g5 = gvec[5]
            iw = jnp.maximum(jnp.minimum(x2, g3) - jnp.maximum(x1, g0) + 1.0, 0.0)
            ih = jnp.maximum(jnp.minimum(y2, g4) - jnp.maximum(y1, g1) + 1.0, 0.0)
            it = jnp.maximum(jnp.minimum(t2, g5) - jnp.maximum(t1, g2) + 1.0, 0.0)
            inter = iw * ih * it
            volg = (g3 - g0 + 1.0) * (g4 - g1 + 1.0) * (g5 - g2 + 1.0)
            iou = inter / (vol_r + volg - inter)
            gv = jnp.full((L,), 0, jnp.int32) + g
            flag = jnp.full((L,), 0.0, jnp.float32) + gvec[7]
            val = jnp.where(flag > 0.5, iou, -1.0)
            upd = val > mv
            mv = jnp.where(upd, val, mv)
            av = jnp.where(upd, gv, av)
            return mv, av

        mv0 = jnp.full((L,), -1.0, jnp.float32)
        av0 = jnp.zeros((L,), jnp.int32)
        mv, av = lax.fori_loop(0, G, g_body, (mv0, av0))
        s_max_c[pl.ds(v * L, L)] = mv
        s_asn_c[pl.ds(v * L, L)] = av
        return 0

    lax.fori_loop(0, NV_CHUNK, v_body, 0)
    pltpu.sync_copy(s_max_c, sp_max.at[pl.ds(bl * N_PAD + base, CHUNK)])
    pltpu.sync_copy(s_asn_c, sp_asn.at[pl.ds(bl * N_PAD + base, CHUNK)])
    plsc.subcore_barrier()

    @pl.when(ch == 0)
    def _leader():
        xor_idx = [jnp.bitwise_xor(iota, sh) for sh in (8, 4, 2, 1)]

        def _vmax(x):
            for ix in xor_idx:
                s_tmp[pl.ds(0, L)] = x
                x = jnp.maximum(x, plsc.load_gather(s_tmp, [ix]))
            return x[0]

        def _vmin(x):
            for ix in xor_idx:
                s_tmp[pl.ds(0, L)] = x
                x = jnp.minimum(x, plsc.load_gather(s_tmp, [ix]))
            return x[0]

        pltpu.sync_copy(sp_max.at[pl.ds(bl * N_PAD, N_PAD)], s_max)
        pltpu.sync_copy(sp_asn.at[pl.ds(bl * N_PAD, N_PAD)], s_asn)
        pltpu.sync_copy(coords_hbm.at[pl.ds(b * (6 * N_PAD), 6 * N_PAD)],
                        s_coord)

        def sc_body(v, _):
            m = s_max[pl.ds(v * L, L)]
            fg = jnp.where(m >= FG_TH, m, -1.0)
            bg = jnp.where((m >= BG_LO) & (m < BG_HI), 1.0 - m, -1.0)
            s_fg[pl.ds(v * L, L)] = fg
            s_bg[pl.ds(v * L, L)] = bg
            return 0
        lax.fori_loop(0, NV_FULL, sc_body, 0)
        pad = jnp.full((L,), -2.0, jnp.float32)
        s_fg[pl.ds(N_TOT, L)] = pad
        s_bg[pl.ds(N_TOT, L)] = pad

        padl1 = jnp.full((L,), -3.0, jnp.float32)
        s_l1fg[pl.ds(NL1 - L, L)] = padl1
        s_l1bg[pl.ds(NL1 - L, L)] = padl1

        def l1_body(v, _):
            put1(s_l1fg, v, _vmax(s_fg[pl.ds(v * L, L)]))
            put1(s_l1bg, v, _vmax(s_bg[pl.ds(v * L, L)]))
            return 0
        lax.fori_loop(0, NV_FULL, l1_body, 0)

        def select(score_ref, l1_ref, k, off, is_fg):
            def it_body(i, _):
                mv = l1_ref[pl.ds(0, L)]
                for vv in range(1, NL1 // L):
                    mv = jnp.maximum(mv, l1_ref[pl.ds(vv * L, L)])
                m_val = _vmax(mv)
                iotaf = iota.astype(jnp.float32)
                bigf = jnp.float32(BIG)
                pmin = jnp.full((L,), BIG, jnp.float32)
                for vv in range(NL1 // L):
                    lv = l1_ref[pl.ds(vv * L, L)]
                    pmin = jnp.minimum(
                        pmin, jnp.where(lv == m_val, vv * L + iotaf, bigf))
                vstar = _vmin(pmin).astype(jnp.int32)
                sv = score_ref[pl.ds(vstar * L, L)]
                lstar = _vmin(
                    jnp.where(sv == m_val, iotaf, bigf)).astype(jnp.int32)
                put1(s_selidx, off + i, vstar * L + lstar)
                if is_fg:
                    put1(s_slotfg, off + i,
                         jnp.where(m_val >= FG_TH, 1.0, 0.0))
                else:
                    put1(s_slotfg, off + i, jnp.float32(0.0))
                sv2 = jnp.where(iota == lstar, jnp.float32(-3.0), sv)
                score_ref[pl.ds(vstar * L, L)] = sv2
                put1(l1_ref, vstar, _vmax(sv2))
                return 0
            lax.fori_loop(0, k, it_body, 0)

        select(s_fg, s_l1fg, K_FG, 0, True)
        select(s_bg, s_l1bg, K_BG, K_FG, False)

        bf = b.astype(jnp.float32)
        for vec in range(K_ALL // L):
            idxv = s_selidx[pl.ds(vec * L, L)]
            slotv = vec * L + iota
            fgm = s_slotfg[pl.ds(vec * L, L)] > 0.5
            ga = plsc.load_gather(s_asn, [idxv])
            r = [plsc.load_gather(s_coord, [idxv + j * N_PAD]) for j in range(6)]
            g = [plsc.load_gather(s_gt, [ga * 8 + j]) for j in range(7)]
            ex_w = r[3] - r[0] + 1.0
            ex_h = r[4] - r[1] + 1.0
            ex_d = r[5] - r[2] + 1.0
            ex_cx = r[0] + 0.5 * ex_w
            ex_cy = r[1] + 0.5 * ex_h
            ex_ct = r[2] + 0.5 * ex_d
            gt_w = g[3] - g[0] + 1.0
            gt_h = g[4] - g[1] + 1.0
            gt_d = g[5] - g[2] + 1.0
            gt_cx = g[0] + 0.5 * gt_w
            gt_cy = g[1] + 0.5 * gt_h
            gt_ct = g[2] + 0.5 * gt_d
            t6 = [(gt_cx - ex_cx) / ex_w,
                  (gt_cy - ex_cy) / ex_h,
                  (gt_ct - ex_ct) / ex_d,
                  _ln(gt_w / ex_w),
                  _ln(gt_h / ex_h),
                  _ln(gt_d / ex_d)]
            lab = jnp.where(fgm, g[6], 0.0)
            wv = jnp.where(fgm, 1.0, 0.0)
            s_lab[pl.ds(vec * L, L)] = lab
            for j in range(6):
                tj = jnp.where(fgm, t6[j] / STDS[j], 0.0)
                plsc.store_scatter(s_tgt, [slotv * 6 + j], tj)
                plsc.store_scatter(s_w, [slotv * 6 + j], wv)
            plsc.store_scatter(s_rois, [slotv * 7],
                               jnp.zeros((L,), jnp.float32) + bf)
            for j in range(6):
                plsc.store_scatter(s_rois, [slotv * 7 + (j + 1)], r[j])
        pltpu.sync_copy(s_rois, rois_out.at[pl.ds(b * (K_ALL * 7), K_ALL * 7)])
        pltpu.sync_copy(s_lab, lab_out.at[pl.ds(b * K_ALL, K_ALL)])
        pltpu.sync_copy(s_tgt, tgt_out.at[pl.ds(b * (K_ALL * 6), K_ALL * 6)])
        pltpu.sync_copy(s_w, w_out.at[pl.ds(b * (K_ALL * 6), K_ALL * 6)])


def _build():
    return pl.kernel(
        _body,
        out_type=[
            jax.ShapeDtypeStruct((B * K_ALL * 7,), jnp.float32),
            jax.ShapeDtypeStruct((B * K_ALL,), jnp.float32),
            jax.ShapeDtypeStruct((B * K_ALL * 6,), jnp.float32),
            jax.ShapeDtypeStruct((B * K_ALL * 6,), jnp.float32),
        ],
        mesh=plsc.VectorSubcoreMesh(core_axis_name="c", subcore_axis_name="s"),
        compiler_params=pltpu.CompilerParams(needs_layout_passes=False),
        scratch_types=[
            pltpu.VMEM((6 * CHUNK,), jnp.float32),
            pltpu.VMEM((CHUNK,), jnp.float32),
            pltpu.VMEM((CHUNK,), jnp.int32),
            pltpu.VMEM_SHARED((4 * N_PAD,), jnp.float32),
            pltpu.VMEM_SHARED((4 * N_PAD,), jnp.int32),
            pltpu.VMEM((6 * N_PAD,), jnp.float32),
            pltpu.VMEM((GT_FLAT,), jnp.float32),
            pltpu.VMEM((N_PAD,), jnp.float32),
            pltpu.VMEM((N_PAD,), jnp.int32),
            pltpu.VMEM((N_PAD,), jnp.float32),
            pltpu.VMEM((N_PAD,), jnp.float32),
            pltpu.VMEM((NL1,), jnp.float32),
            pltpu.VMEM((NL1,), jnp.float32),
            pltpu.VMEM((K_ALL,), jnp.int32),
            pltpu.VMEM((K_ALL,), jnp.float32),
            pltpu.VMEM((K_ALL * 7,), jnp.float32),
            pltpu.VMEM((K_ALL,), jnp.float32),
            pltpu.VMEM((K_ALL * 6,), jnp.float32),
            pltpu.VMEM((K_ALL * 6,), jnp.float32),
            pltpu.VMEM((L,), jnp.float32),
        ],
    )


_SC_FN = None


def _sc_fn():
    global _SC_FN
    if _SC_FN is None:
        _SC_FN = _build()
    return _SC_FN


def kernel(all_tubes, gt_boxes, num_boxes):
    nb = num_boxes.astype(jnp.int32)
    valid = (jnp.arange(G, dtype=jnp.int32)[None, :] < nb[:, None])
    gt6 = gt_boxes[:, :, :6] * valid[..., None].astype(jnp.float32)
    rois6 = jnp.concatenate([all_tubes[:, :, 1:7], gt6], axis=1)
    coords_t = jnp.transpose(rois6, (0, 2, 1))
    coords_t = jnp.pad(coords_t, ((0, 0), (0, 0), (0, N_PAD - N_TOT)))
    coords_flat = coords_t.reshape(B * 6 * N_PAD)
    gt_tab = jnp.concatenate([gt6, gt_boxes[:, :, 6:7],
                              valid[..., None].astype(jnp.float32)], axis=-1)
    gt_flat = gt_tab.reshape(B * G * 8)
    rois, labels, targets, w = _sc_fn()(coords_flat, gt_flat)
    rois = rois.reshape(B, K_ALL, 7)
    labels = labels.reshape(B, K_ALL)
    targets = targets.reshape(B, K_ALL, 6)
    w = w.reshape(B, K_ALL, 6)
    return rois, labels, targets, w, w
